# Initial kernel scaffold; baseline (speedup 1.0000x reference)
import functools

import jax
import jax.numpy as jnp
from jax.experimental import pallas as pl
from jax.experimental.pallas import tpu as pltpu

H = 128
DEPTH = 17
N = 2**DEPTH - 1
ANY = pltpu.MemorySpace.ANY
VMEM = pltpu.MemorySpace.VMEM
F32 = jnp.float32


def _internal_math(wx, h2, Urt, Uhct, Uzt):
    e = h2[:, :H]
    o = h2[:, H:]
    r = jax.nn.sigmoid(wx[:, :H] + jnp.dot(h2, Urt, preferred_element_type=F32))
    rh = jnp.concatenate([r, r], axis=1) * h2
    hcand = jnp.tanh(
        wx[:, H:2 * H] + jnp.dot(rh, Uhct, preferred_element_type=F32))
    z = jax.nn.sigmoid(
        wx[:, 2 * H:] + jnp.dot(h2, Uzt, preferred_element_type=F32))
    z0 = z[:, :H]
    z1 = z[:, H:]
    return e * z0 + o * z1 + (1.0 - z0 - z1) * hcand


def _leaf_math(wx):
    hcand = jnp.tanh(wx[:, :H])
    z0 = jax.nn.sigmoid(wx[:, H:2 * H])
    z1 = jax.nn.sigmoid(wx[:, 2 * H:])
    return (1.0 - z0 - z1) * hcand


def _leaf_body(x_hbm, Wtl, Wbl, h_out, outf_out, xbuf, obuf, sx, so,
               *, start, B, nsteps):
    i = pl.program_id(0)
    slot = jax.lax.rem(i, 2)

    def x_copy(step, s):
        return pltpu.make_async_copy(
            x_hbm.at[pl.ds(start + step * B, B)], xbuf.at[s], sx.at[s])

    def o_copy(step, s):
        return pltpu.make_async_copy(
            obuf.at[s], outf_out.at[pl.ds(start + step * B, B)], so.at[s])

    @pl.when(i == 0)
    def _():
        x_copy(0, 0).start()

    @pl.when(i + 1 < nsteps)
    def _():
        x_copy(i + 1, 1 - slot).start()

    x_copy(i, slot).wait()

    wx = jnp.dot(xbuf[slot], Wtl[...], preferred_element_type=F32) + Wbl[...]
    hnew = _leaf_math(wx)
    h_out[...] = hnew

    @pl.when(i >= 2)
    def _():
        o_copy(i - 2, slot).wait()

    obuf[slot] = hnew
    o_copy(i, slot).start()

    @pl.when(i == nsteps - 1)
    def _():
        if nsteps >= 2:
            o_copy(nsteps - 2, (nsteps - 2) % 2).wait()
        o_copy(nsteps - 1, (nsteps - 1) % 2).wait()


def _internal_body(x_hbm, hpair, outf_in, Wt, Wb, Urt, Uhct, Uzt,
                   h_out, outf_out, xbuf, obuf, sx, so,
                   *, start, B, nsteps):
    i = pl.program_id(0)
    slot = jax.lax.rem(i, 2)

    def x_copy(step, s):
        return pltpu.make_async_copy(
            x_hbm.at[pl.ds(start + step * B, B)], xbuf.at[s], sx.at[s])

    def o_copy(step, s):
        return pltpu.make_async_copy(
            obuf.at[s], outf_out.at[pl.ds(start + step * B, B)], so.at[s])

    @pl.when(i == 0)
    def _():
        x_copy(0, 0).start()

    @pl.when(i + 1 < nsteps)
    def _():
        x_copy(i + 1, 1 - slot).start()

    x_copy(i, slot).wait()

    wx = jnp.dot(xbuf[slot], Wt[...], preferred_element_type=F32) + Wb[...]
    hnew = _internal_math(wx, hpair[...], Urt[...], Uhct[...], Uzt[...])
    h_out[...] = hnew

    @pl.when(i >= 2)
    def _():
        o_copy(i - 2, slot).wait()

    obuf[slot] = hnew
    o_copy(i, slot).start()

    @pl.when(i == nsteps - 1)
    def _():
        if nsteps >= 2:
            o_copy(nsteps - 2, (nsteps - 2) % 2).wait()
        o_copy(nsteps - 1, (nsteps - 1) % 2).wait()


def _small_body(xs, hpair, outf_in, Wt, Wb, Urt, Uhct, Uzt, outf_out,
                obuf, so, *, top_level):
    h2 = hpair[...]
    n_small = 2 ** (top_level + 1) - 1
    for l in range(top_level, -1, -1):
        L = 2 ** l
        start = L - 1
        xb = xs[pl.ds(start, L), :]
        wx = jnp.dot(xb, Wt[...], preferred_element_type=F32) + Wb[...]
        hnew = _internal_math(wx, h2, Urt[...], Uhct[...], Uzt[...])
        obuf[pl.ds(start, L), :] = hnew
        if l > 0:
            Lp = L // 2
            rows = jax.lax.broadcasted_iota(jnp.int32, (Lp, L), 0)
            cols = jax.lax.broadcasted_iota(jnp.int32, (Lp, L), 1)
            sel_e = (cols == 2 * rows).astype(F32)
            sel_o = (cols == 2 * rows + 1).astype(F32)
            e_next = jnp.dot(sel_e, hnew, preferred_element_type=F32)
            o_next = jnp.dot(sel_o, hnew, preferred_element_type=F32)
            h2 = jnp.concatenate([e_next, o_next], axis=1)
    cp = pltpu.make_async_copy(
        obuf.at[pl.ds(0, n_small)], outf_out.at[pl.ds(0, n_small)], so)
    cp.start()
    cp.wait()


def kernel(x, W_w, W_b, U_r, U_hc, U_z):
    Wt = W_w.T.astype(F32)
    Wb = W_b.reshape(1, -1).astype(F32)
    Urt = U_r.T.astype(F32)
    Uhct = U_hc.T.astype(F32)
    Uzt = U_z.T.astype(F32)
    Wtl = Wt[:, H:]
    Wbl = Wb[:, H:]

    B = 2048
    TOP = 10

    L = 2 ** (DEPTH - 1)
    ns = L // B
    h_lvl, outf = pl.pallas_call(
        functools.partial(_leaf_body, start=L - 1, B=B, nsteps=ns),
        grid=(ns,),
        in_specs=[
            pl.BlockSpec(memory_space=ANY),
            pl.BlockSpec(memory_space=VMEM),
            pl.BlockSpec(memory_space=VMEM),
        ],
        out_specs=[
            pl.BlockSpec((B, H), lambda i: (i, 0)),
            pl.BlockSpec(memory_space=ANY),
        ],
        out_shape=[
            jax.ShapeDtypeStruct((L, H), F32),
            jax.ShapeDtypeStruct((N, H), F32),
        ],
        scratch_shapes=[
            pltpu.VMEM((2, B, H), F32),
            pltpu.VMEM((2, B, H), F32),
            pltpu.SemaphoreType.DMA((2,)),
            pltpu.SemaphoreType.DMA((2,)),
        ],
    )(x, Wtl, Wbl)

    for l in range(DEPTH - 2, TOP, -1):
        L = 2 ** l
        Bl = min(B, L)
        ns = L // Bl
        hpair = h_lvl.reshape(L, 2 * H)
        h_lvl, outf = pl.pallas_call(
            functools.partial(_internal_body, start=L - 1, B=Bl, nsteps=ns),
            grid=(ns,),
            in_specs=[
                pl.BlockSpec(memory_space=ANY),
                pl.BlockSpec((Bl, 2 * H), lambda i: (i, 0)),
                pl.BlockSpec(memory_space=ANY),
                pl.BlockSpec(memory_space=VMEM),
                pl.BlockSpec(memory_space=VMEM),
                pl.BlockSpec(memory_space=VMEM),
                pl.BlockSpec(memory_space=VMEM),
                pl.BlockSpec(memory_space=VMEM),
            ],
            out_specs=[
                pl.BlockSpec((Bl, H), lambda i: (i, 0)),
                pl.BlockSpec(memory_space=ANY),
            ],
            out_shape=[
                jax.ShapeDtypeStruct((L, H), F32),
                jax.ShapeDtypeStruct((N, H), F32),
            ],
            scratch_shapes=[
                pltpu.VMEM((2, Bl, H), F32),
                pltpu.VMEM((2, Bl, H), F32),
                pltpu.SemaphoreType.DMA((2,)),
                pltpu.SemaphoreType.DMA((2,)),
            ],
            input_output_aliases={2: 1},
        )(x, hpair, outf, Wt, Wb, Urt, Uhct, Uzt)

    n_small = 2 ** (TOP + 1) - 1
    xs = jax.lax.slice(x, (0, 0), (n_small, H))
    hpair = h_lvl.reshape(2 ** TOP, 2 * H)
    outf = pl.pallas_call(
        functools.partial(_small_body, top_level=TOP),
        in_specs=[
            pl.BlockSpec(memory_space=VMEM),
            pl.BlockSpec(memory_space=VMEM),
            pl.BlockSpec(memory_space=ANY),
            pl.BlockSpec(memory_space=VMEM),
            pl.BlockSpec(memory_space=VMEM),
            pl.BlockSpec(memory_space=VMEM),
            pl.BlockSpec(memory_space=VMEM),
            pl.BlockSpec(memory_space=VMEM),
        ],
        out_specs=pl.BlockSpec(memory_space=ANY),
        out_shape=jax.ShapeDtypeStruct((N, H), F32),
        scratch_shapes=[
            pltpu.VMEM((2 ** (TOP + 1), H), F32),
            pltpu.SemaphoreType.DMA,
        ],
        input_output_aliases={2: 0},
    )(xs, hpair, outf, Wt, Wb, Urt, Uhct, Uzt)
    return outf


# speedup vs baseline: 19.6933x; 19.6933x over previous
import functools

import jax
import jax.numpy as jnp
from jax.experimental import pallas as pl
from jax.experimental.pallas import tpu as pltpu

H = 128
DEPTH = 17
N = 2**DEPTH - 1
ANY = pl.ANY
VMEM = pltpu.MemorySpace.VMEM
F32 = jnp.float32


def _internal_math(wx, h2, Urt, Uhct, Uzt):
    e = h2[:, :H]
    o = h2[:, H:]
    r = jax.nn.sigmoid(wx[:, :H] + jnp.dot(h2, Urt, preferred_element_type=F32))
    rh = jnp.concatenate([r, r], axis=1) * h2
    hcand = jnp.tanh(
        wx[:, H:2 * H] + jnp.dot(rh, Uhct, preferred_element_type=F32))
    z = jax.nn.sigmoid(
        wx[:, 2 * H:] + jnp.dot(h2, Uzt, preferred_element_type=F32))
    z0 = z[:, :H]
    z1 = z[:, H:]
    return e * z0 + o * z1 + (1.0 - z0 - z1) * hcand


def _leaf_math(wx):
    hcand = jnp.tanh(wx[:, :H])
    z0 = jax.nn.sigmoid(wx[:, H:2 * H])
    z1 = jax.nn.sigmoid(wx[:, 2 * H:])
    return (1.0 - z0 - z1) * hcand


def _leaf_body(x_hbm, Wtl, Wbl, h_out, outf_out, xbuf, obuf, sx, so,
               *, start, B, nsteps):
    i = pl.program_id(0)
    slot = jax.lax.rem(i, 2)

    def x_copy(step, s):
        return pltpu.make_async_copy(
            x_hbm.at[pl.ds(start + step * B, B)], xbuf.at[s], sx.at[s])

    def o_copy(step, s):
        return pltpu.make_async_copy(
            obuf.at[s], outf_out.at[pl.ds(start + step * B, B)], so.at[s])

    @pl.when(i == 0)
    def _():
        x_copy(0, 0).start()

    @pl.when(i + 1 < nsteps)
    def _():
        x_copy(i + 1, 1 - slot).start()

    x_copy(i, slot).wait()

    wx = jnp.dot(xbuf[slot], Wtl[...], preferred_element_type=F32) + Wbl[...]
    hnew = _leaf_math(wx)
    h_out[...] = hnew

    @pl.when(i >= 2)
    def _():
        o_copy(i - 2, slot).wait()

    obuf[slot] = hnew
    o_copy(i, slot).start()

    @pl.when(i == nsteps - 1)
    def _():
        if nsteps >= 2:
            o_copy(nsteps - 2, (nsteps - 2) % 2).wait()
        o_copy(nsteps - 1, (nsteps - 1) % 2).wait()


def _internal_body(x_hbm, hpair, outf_in, Wt, Wb, Urt, Uhct, Uzt,
                   h_out, outf_out, xbuf, obuf, sx, so,
                   *, start, B, nsteps):
    i = pl.program_id(0)
    slot = jax.lax.rem(i, 2)

    def x_copy(step, s):
        return pltpu.make_async_copy(
            x_hbm.at[pl.ds(start + step * B, B)], xbuf.at[s], sx.at[s])

    def o_copy(step, s):
        return pltpu.make_async_copy(
            obuf.at[s], outf_out.at[pl.ds(start + step * B, B)], so.at[s])

    @pl.when(i == 0)
    def _():
        x_copy(0, 0).start()

    @pl.when(i + 1 < nsteps)
    def _():
        x_copy(i + 1, 1 - slot).start()

    x_copy(i, slot).wait()

    wx = jnp.dot(xbuf[slot], Wt[...], preferred_element_type=F32) + Wb[...]
    hnew = _internal_math(wx, hpair[...], Urt[...], Uhct[...], Uzt[...])
    h_out[...] = hnew

    @pl.when(i >= 2)
    def _():
        o_copy(i - 2, slot).wait()

    obuf[slot] = hnew
    o_copy(i, slot).start()

    @pl.when(i == nsteps - 1)
    def _():
        if nsteps >= 2:
            o_copy(nsteps - 2, (nsteps - 2) % 2).wait()
        o_copy(nsteps - 1, (nsteps - 1) % 2).wait()


def _small_body(xs, hpair, outf_in, Wt, Wb, Urt, Uhct, Uzt, outf_out,
                obuf, so, *, top_level):
    h2 = hpair[...]
    n_small = 2 ** (top_level + 1) - 1
    for l in range(top_level, -1, -1):
        L = 2 ** l
        start = L - 1
        xb = xs[pl.ds(start, L), :]
        wx = jnp.dot(xb, Wt[...], preferred_element_type=F32) + Wb[...]
        hnew = _internal_math(wx, h2, Urt[...], Uhct[...], Uzt[...])
        obuf[pl.ds(start, L), :] = hnew
        if l > 0:
            Lp = L // 2
            rows = jax.lax.broadcasted_iota(jnp.int32, (Lp, L), 0)
            cols = jax.lax.broadcasted_iota(jnp.int32, (Lp, L), 1)
            sel_e = (cols == 2 * rows).astype(F32)
            sel_o = (cols == 2 * rows + 1).astype(F32)
            e_next = jnp.dot(sel_e, hnew, preferred_element_type=F32)
            o_next = jnp.dot(sel_o, hnew, preferred_element_type=F32)
            h2 = jnp.concatenate([e_next, o_next], axis=1)
    cp = pltpu.make_async_copy(
        obuf.at[pl.ds(0, n_small)], outf_out.at[pl.ds(0, n_small)], so)
    cp.start()
    cp.wait()


def kernel(x, W_w, W_b, U_r, U_hc, U_z):
    Wt = W_w.T.astype(F32)
    Wb = W_b.reshape(1, -1).astype(F32)
    Urt = U_r.T.astype(F32)
    Uhct = U_hc.T.astype(F32)
    Uzt = U_z.T.astype(F32)
    Wtl = Wt[:, H:]
    Wbl = Wb[:, H:]

    B = 2048
    TOP = 10

    L = 2 ** (DEPTH - 1)
    ns = L // B
    h_lvl, outf = pl.pallas_call(
        functools.partial(_leaf_body, start=L - 1, B=B, nsteps=ns),
        grid=(ns,),
        in_specs=[
            pl.BlockSpec(memory_space=ANY),
            pl.BlockSpec(memory_space=VMEM),
            pl.BlockSpec(memory_space=VMEM),
        ],
        out_specs=[
            pl.BlockSpec((B, H), lambda i: (i, 0)),
            pl.BlockSpec(memory_space=ANY),
        ],
        out_shape=[
            jax.ShapeDtypeStruct((L, H), F32),
            jax.ShapeDtypeStruct((N, H), F32),
        ],
        scratch_shapes=[
            pltpu.VMEM((2, B, H), F32),
            pltpu.VMEM((2, B, H), F32),
            pltpu.SemaphoreType.DMA((2,)),
            pltpu.SemaphoreType.DMA((2,)),
        ],
    )(x, Wtl, Wbl)

    for l in range(DEPTH - 2, TOP, -1):
        L = 2 ** l
        Bl = min(B, L)
        ns = L // Bl
        hpair = h_lvl.reshape(L, 2 * H)
        h_lvl, outf = pl.pallas_call(
            functools.partial(_internal_body, start=L - 1, B=Bl, nsteps=ns),
            grid=(ns,),
            in_specs=[
                pl.BlockSpec(memory_space=ANY),
                pl.BlockSpec((Bl, 2 * H), lambda i: (i, 0)),
                pl.BlockSpec(memory_space=ANY),
                pl.BlockSpec(memory_space=VMEM),
                pl.BlockSpec(memory_space=VMEM),
                pl.BlockSpec(memory_space=VMEM),
                pl.BlockSpec(memory_space=VMEM),
                pl.BlockSpec(memory_space=VMEM),
            ],
            out_specs=[
                pl.BlockSpec((Bl, H), lambda i: (i, 0)),
                pl.BlockSpec(memory_space=ANY),
            ],
            out_shape=[
                jax.ShapeDtypeStruct((L, H), F32),
                jax.ShapeDtypeStruct((N, H), F32),
            ],
            scratch_shapes=[
                pltpu.VMEM((2, Bl, H), F32),
                pltpu.VMEM((2, Bl, H), F32),
                pltpu.SemaphoreType.DMA((2,)),
                pltpu.SemaphoreType.DMA((2,)),
            ],
            input_output_aliases={2: 1},
        )(x, hpair, outf, Wt, Wb, Urt, Uhct, Uzt)

    n_small = 2 ** (TOP + 1) - 1
    xs = jax.lax.slice(x, (0, 0), (n_small, H))
    hpair = h_lvl.reshape(2 ** TOP, 2 * H)
    outf = pl.pallas_call(
        functools.partial(_small_body, top_level=TOP),
        in_specs=[
            pl.BlockSpec(memory_space=VMEM),
            pl.BlockSpec(memory_space=VMEM),
            pl.BlockSpec(memory_space=ANY),
            pl.BlockSpec(memory_space=VMEM),
            pl.BlockSpec(memory_space=VMEM),
            pl.BlockSpec(memory_space=VMEM),
            pl.BlockSpec(memory_space=VMEM),
            pl.BlockSpec(memory_space=VMEM),
        ],
        out_specs=pl.BlockSpec(memory_space=ANY),
        out_shape=jax.ShapeDtypeStruct((N, H), F32),
        scratch_shapes=[
            pltpu.VMEM((2 ** (TOP + 1), H), F32),
            pltpu.SemaphoreType.DMA,
        ],
        input_output_aliases={2: 0},
    )(xs, hpair, outf, Wt, Wb, Urt, Uhct, Uzt)
    return outf


# speedup vs baseline: 33.5847x; 1.7054x over previous
import functools

import jax
import jax.numpy as jnp
from jax.experimental import pallas as pl
from jax.experimental.pallas import tpu as pltpu

H = 128
DEPTH = 17
N = 2**DEPTH - 1
B = 2048
TOP = 10
NBIG = DEPTH - 1 - TOP
NSTEPS = 2 ** NBIG
N_SMALL = 2 ** (TOP + 1) - 1
ANY = pl.ANY
VMEM = pltpu.MemorySpace.VMEM
SMEM = pltpu.MemorySpace.SMEM
F32 = jnp.float32
BF16 = jnp.bfloat16

_REGION = {DEPTH - 1 - k: 2 * B * k for k in range(NBIG)}
_HBUF_ROWS = 2 * B * NBIG


def _schedule():
    def post(l, m):
        if l == DEPTH - 1:
            return [(l, m)]
        return post(l + 1, 2 * m) + post(l + 1, 2 * m + 1) + [(l, m)]

    seq = post(TOP + 1, 0)
    typ, xs, cb, wo = [], [], [], []
    for (l, m) in seq:
        typ.append(0 if l == DEPTH - 1 else 1)
        xs.append(2**l - 1 + m * B)
        cb.append(_REGION[l + 1] if l < DEPTH - 1 else 0)
        wo.append(_REGION[l] + (m % 2) * B)
    typ.append(2)
    xs.append(0)
    cb.append(_REGION[TOP + 1])
    wo.append(0)
    return typ, xs, cb, wo


def _internal_math(wx, h2, Urt, Uhct, Uzt, mm_dtype=F32):
    e = h2[:, :H]
    o = h2[:, H:]
    h2m = h2.astype(mm_dtype)
    r = jax.nn.sigmoid(wx[:, :H] + jnp.dot(h2m, Urt, preferred_element_type=F32))
    rh = (jnp.concatenate([r, r], axis=1) * h2).astype(mm_dtype)
    hcand = jnp.tanh(
        wx[:, H:2 * H] + jnp.dot(rh, Uhct, preferred_element_type=F32))
    z = jax.nn.sigmoid(
        wx[:, 2 * H:] + jnp.dot(h2m, Uzt, preferred_element_type=F32))
    z0 = z[:, :H]
    z1 = z[:, H:]
    return e * z0 + o * z1 + (1.0 - z0 - z1) * hcand


def _leaf_math(wx):
    hcand = jnp.tanh(wx[:, :H])
    z0 = jax.nn.sigmoid(wx[:, H:2 * H])
    z1 = jax.nn.sigmoid(wx[:, 2 * H:])
    return (1.0 - z0 - z1) * hcand


def _body(x_hbm, t_typ, t_xs, t_cb, t_wo,
          Wtl_b, Wbl, Wt_b, Wb, Urt_b, Uhct_b, Uzt_b,
          Wt_f, Urt_f, Uhct_f, Uzt_f,
          outf, xbuf, obuf, hbuf, sbuf, sx, so, ss):
    i = pl.program_id(0)
    slot = jax.lax.rem(i, 2)

    def x_copy(row0, s):
        return pltpu.make_async_copy(
            x_hbm.at[pl.ds(row0, B)], xbuf.at[s], sx.at[s])

    def out_copy(row0, s):
        return pltpu.make_async_copy(
            obuf.at[s], outf.at[pl.ds(row0, B)], so.at[s])

    @pl.when(i == 0)
    def _():
        x_copy(t_xs[0], 0).start()

    @pl.when(i + 1 < NSTEPS)
    def _():
        x_copy(t_xs[i + 1], 1 - slot).start()

    x_copy(t_xs[i], slot).wait()

    typ = t_typ[i]
    xs_i = t_xs[i]
    cb = t_cb[i]
    wo = t_wo[i]

    @pl.when(i >= 2)
    def _():
        out_copy(0, slot).wait()

    @pl.when(typ == 0)
    def _():
        xb = xbuf[slot].astype(BF16)
        wx = jnp.dot(xb, Wtl_b[...], preferred_element_type=F32) + Wbl[...]
        hnew = _leaf_math(wx)
        hbuf[pl.ds(wo, B), :] = hnew
        obuf[slot] = hnew

    @pl.when(typ == 1)
    def _():
        xb = xbuf[slot].astype(BF16)
        wx = jnp.dot(xb, Wt_b[...], preferred_element_type=F32) + Wb[...]
        h2 = hbuf[pl.ds(cb, 2 * B), :].reshape(B, 2 * H)
        hnew = _internal_math(wx, h2, Urt_b[...], Uhct_b[...], Uzt_b[...],
                              mm_dtype=BF16)
        hbuf[pl.ds(wo, B), :] = hnew
        obuf[slot] = hnew

    @pl.when(i < NSTEPS - 1)
    def _():
        out_copy(xs_i, slot).start()

    @pl.when(typ == 2)
    def _():
        out_copy(0, 1 - slot).wait()
        xsm = xbuf[slot]
        h2 = hbuf[pl.ds(cb, 2 ** (TOP + 1)), :].reshape(2 ** TOP, 2 * H)
        for l in range(TOP, -1, -1):
            L = 2 ** l
            start = L - 1
            xb = jax.lax.slice(xsm, (start, 0), (start + L, H))
            wx = jnp.dot(xb, Wt_f[...], preferred_element_type=F32) + Wb[...]
            hnew = _internal_math(wx, h2, Urt_f[...], Uhct_f[...], Uzt_f[...])
            sbuf[pl.ds(start, L), :] = hnew
            if l > 0:
                h2 = hnew.reshape(L // 2, 2 * H)
        cp = pltpu.make_async_copy(
            sbuf.at[pl.ds(0, N_SMALL)], outf.at[pl.ds(0, N_SMALL)], ss)
        cp.start()
        cp.wait()


def kernel(x, W_w, W_b, U_r, U_hc, U_z):
    Wt = W_w.T.astype(F32)
    Wb = W_b.reshape(1, -1).astype(F32)
    Urt = U_r.T.astype(F32)
    Uhct = U_hc.T.astype(F32)
    Uzt = U_z.T.astype(F32)
    Wtl_b = Wt[:, H:].astype(BF16)
    Wbl = Wb[:, H:]
    Wt_b = Wt.astype(BF16)
    Urt_b = Urt.astype(BF16)
    Uhct_b = Uhct.astype(BF16)
    Uzt_b = Uzt.astype(BF16)

    typ, xs, cb, wo = _schedule()
    t_typ = jnp.asarray(typ, jnp.int32)
    t_xs = jnp.asarray(xs, jnp.int32)
    t_cb = jnp.asarray(cb, jnp.int32)
    t_wo = jnp.asarray(wo, jnp.int32)

    outf = pl.pallas_call(
        _body,
        grid=(NSTEPS,),
        in_specs=[
            pl.BlockSpec(memory_space=ANY),
            pl.BlockSpec(memory_space=SMEM),
            pl.BlockSpec(memory_space=SMEM),
            pl.BlockSpec(memory_space=SMEM),
            pl.BlockSpec(memory_space=SMEM),
            pl.BlockSpec(memory_space=VMEM),
            pl.BlockSpec(memory_space=VMEM),
            pl.BlockSpec(memory_space=VMEM),
            pl.BlockSpec(memory_space=VMEM),
            pl.BlockSpec(memory_space=VMEM),
            pl.BlockSpec(memory_space=VMEM),
            pl.BlockSpec(memory_space=VMEM),
            pl.BlockSpec(memory_space=VMEM),
            pl.BlockSpec(memory_space=VMEM),
            pl.BlockSpec(memory_space=VMEM),
            pl.BlockSpec(memory_space=VMEM),
        ],
        out_specs=pl.BlockSpec(memory_space=ANY),
        out_shape=jax.ShapeDtypeStruct((N, H), F32),
        scratch_shapes=[
            pltpu.VMEM((2, B, H), F32),
            pltpu.VMEM((2, B, H), F32),
            pltpu.VMEM((_HBUF_ROWS, H), F32),
            pltpu.VMEM((2 ** (TOP + 1), H), F32),
            pltpu.SemaphoreType.DMA((2,)),
            pltpu.SemaphoreType.DMA((2,)),
            pltpu.SemaphoreType.DMA,
        ],
    )(x, t_typ, t_xs, t_cb, t_wo,
      Wtl_b, Wbl, Wt_b, Wb, Urt_b, Uhct_b, Uzt_b,
      Wt, Urt, Uhct, Uzt)
    return outf
